# baseline (speedup 1.0000x reference)
import functools

import jax
import jax.numpy as jnp
from jax import lax
from jax.experimental import pallas as pl
from jax.experimental.pallas import tpu as pltpu
from jax.experimental.pallas import tpu_sc as plsc

BATCH = 4096
HIST = 50
D = 128
NC, NS = 2, 16
NW = NC * NS
COLS = BATCH // NW
CHUNK = 64
NBUF = 12
NCHUNKS = 2 * HIST
NLOOP = (NCHUNKS - NBUF) // NBUF
NREM = NCHUNKS - NBUF * (NLOOP + 1)

_mesh = plsc.VectorSubcoreMesh(core_axis_name="c", subcore_axis_name="s")


@functools.partial(
    pl.kernel,
    out_type=jax.ShapeDtypeStruct((HIST, BATCH, D), jnp.float32),
    mesh=_mesh,
    scratch_types=[
        pltpu.VMEM((HIST, COLS), jnp.int32),
        pltpu.VMEM((NBUF, CHUNK, D), jnp.float32),
        [pltpu.SemaphoreType.DMA] * NBUF,
        [pltpu.SemaphoreType.DMA] * NBUF,
    ],
)
def _gather_kernel(table_hbm, idx_hbm, out_hbm, idx_v, rows_v, gsem, osem):
    wid = lax.axis_index("s") * NC + lax.axis_index("c")
    col = wid * COLS
    pltpu.sync_copy(idx_hbm.at[pl.ds(0, HIST), pl.ds(col, COLS)], idx_v)

    def gather(h, half, b):
        pltpu.async_copy(
            table_hbm.at[idx_v.at[h, pl.ds(half * CHUNK, CHUNK)]],
            rows_v.at[b], gsem[b])

    def wait_gather(b):
        pltpu.make_async_copy(
            table_hbm.at[idx_v.at[0, pl.ds(0, CHUNK)]], rows_v.at[b], gsem[b]
        ).wait()

    def put(h, half, b):
        pltpu.async_copy(
            rows_v.at[b], out_hbm.at[h, pl.ds(col + half * CHUNK, CHUNK)],
            osem[b])

    def wait_put(b):
        pltpu.make_async_copy(
            rows_v.at[b], out_hbm.at[0, pl.ds(col, CHUNK)], osem[b]
        ).wait()

    for b in range(NBUF):
        gather(b // 2, b % 2, b)
    for b in range(NBUF):
        wait_gather(b)
        put(b // 2, b % 2, b)

    @pl.loop(1, NLOOP + 1)
    def _group(k):
        h0 = (NBUF // 2) * k
        for b in range(NBUF):
            wait_put(b)
            gather(h0 + b // 2, b % 2, b)
        for b in range(NBUF):
            wait_gather(b)
            put(h0 + b // 2, b % 2, b)

    for r in range(NREM):
        c = NBUF * (NLOOP + 1) + r
        wait_put(r)
        gather(c // 2, c % 2, r)
    for r in range(NREM):
        c = NBUF * (NLOOP + 1) + r
        wait_gather(r)
        put(c // 2, c % 2, r)

    for b in range(NBUF):
        wait_put(b)


def kernel(token_indices, embedding_matrix):
    idx_t = token_indices.T.astype(jnp.int32)
    out_t = _gather_kernel(embedding_matrix, idx_t)
    return out_t.transpose(1, 0, 2)


# speedup vs baseline: 1.0015x; 1.0015x over previous
import functools

import jax
import jax.numpy as jnp
from jax import lax
from jax.experimental import pallas as pl
from jax.experimental.pallas import tpu as pltpu
from jax.experimental.pallas import tpu_sc as plsc

BATCH = 4096
HIST = 50
D = 128
NC, NS = 2, 16
NW = NC * NS
COLS = BATCH // NW
CHUNK = 64
NBUF = 12
NCHUNKS = 2 * HIST
NLOOP = (NCHUNKS - NBUF) // NBUF
NREM = NCHUNKS - NBUF * (NLOOP + 1)

_mesh = plsc.VectorSubcoreMesh(core_axis_name="c", subcore_axis_name="s")


@functools.partial(
    pl.kernel,
    out_type=jax.ShapeDtypeStruct((HIST, BATCH, D), jnp.float32),
    mesh=_mesh,
    scratch_types=[
        pltpu.VMEM((HIST, COLS), jnp.int32),
        pltpu.VMEM((NBUF, CHUNK, D), jnp.float32),
        [pltpu.SemaphoreType.DMA] * NBUF,
        [pltpu.SemaphoreType.DMA] * NBUF,
        pltpu.SemaphoreType.DMA,
    ],
)
def _gather_kernel(table_hbm, idx_hbm, out_hbm, idx_v, rows_v, gsem, osem, isem):
    wid = lax.axis_index("s") * NC + lax.axis_index("c")
    col = wid * COLS
    npro = 8
    pltpu.sync_copy(idx_hbm.at[pl.ds(0, npro), pl.ds(col, COLS)],
                    idx_v.at[pl.ds(0, npro)])
    rest = pltpu.async_copy(
        idx_hbm.at[pl.ds(npro, HIST - npro), pl.ds(col, COLS)],
        idx_v.at[pl.ds(npro, HIST - npro)], isem)

    def gather(h, half, b):
        pltpu.async_copy(
            table_hbm.at[idx_v.at[h, pl.ds(half * CHUNK, CHUNK)]],
            rows_v.at[b], gsem[b])

    def wait_gather(b):
        pltpu.make_async_copy(
            table_hbm.at[idx_v.at[0, pl.ds(0, CHUNK)]], rows_v.at[b], gsem[b]
        ).wait()

    def put(h, half, b):
        pltpu.async_copy(
            rows_v.at[b], out_hbm.at[h, pl.ds(col + half * CHUNK, CHUNK)],
            osem[b])

    def wait_put(b):
        pltpu.make_async_copy(
            rows_v.at[b], out_hbm.at[0, pl.ds(col, CHUNK)], osem[b]
        ).wait()

    for b in range(NBUF):
        gather(b // 2, b % 2, b)
    rest.wait()
    for b in range(NBUF):
        wait_gather(b)
        put(b // 2, b % 2, b)

    @pl.loop(1, NLOOP + 1)
    def _group(k):
        h0 = (NBUF // 2) * k
        for b in range(NBUF):
            wait_put(b)
            gather(h0 + b // 2, b % 2, b)
        for b in range(NBUF):
            wait_gather(b)
            put(h0 + b // 2, b % 2, b)

    for r in range(NREM):
        c = NBUF * (NLOOP + 1) + r
        wait_put(r)
        gather(c // 2, c % 2, r)
    for r in range(NREM):
        c = NBUF * (NLOOP + 1) + r
        wait_gather(r)
        put(c // 2, c % 2, r)

    for b in range(NBUF):
        wait_put(b)


def kernel(token_indices, embedding_matrix):
    idx_t = token_indices.T.astype(jnp.int32)
    out_t = _gather_kernel(embedding_matrix, idx_t)
    return out_t.transpose(1, 0, 2)
